# Initial kernel scaffold; baseline (speedup 1.0000x reference)
import functools

import jax
import jax.numpy as jnp
from jax import lax
from jax.experimental import pallas as pl
from jax.experimental.pallas import tpu as pltpu
from jax.experimental.pallas import tpu_sc as plsc

N = 10000
E = 320000
H = 128
B = 64
NW = 32
EPT = E // NW
CH = 80
NPT = N // 16
EH = E // 2
EHP = 163840
EPTE = EHP // NW
CE = 128

_mesh = plsc.VectorSubcoreMesh(core_axis_name="c", subcore_axis_name="s")


@functools.partial(
    pl.kernel,
    out_type=jax.ShapeDtypeStruct((2 * N, 16), jnp.float32),
    mesh=_mesh,
    scratch_types=[
        pltpu.VMEM((CH,), jnp.int32),
        pltpu.VMEM((CH, 16), jnp.float32),
        pltpu.VMEM((NPT, 16), jnp.float32),
        pltpu.VMEM_SHARED((N, 16), jnp.float32),
    ],
)
def _sc_deg(col_hbm, out_hbm, colv, ones, zb, degsh):
    cid = lax.axis_index("c")
    sid = lax.axis_index("s")

    def _zb(i, _):
        zb[i, :] = jnp.zeros((16,), jnp.float32)
        return 0
    lax.fori_loop(0, NPT, _zb, 0)

    def _ones(i, _):
        ones[i, :] = jnp.full((16,), 1.0, jnp.float32)
        return 0
    lax.fori_loop(0, CH, _ones, 0)

    pltpu.sync_copy(zb, degsh.at[pl.ds(sid * NPT, NPT)])
    plsc.subcore_barrier()

    base = (cid * 16 + sid) * EPT

    def _edges(k, _):
        pltpu.sync_copy(col_hbm.at[pl.ds(base + k * CH, CH)], colv)
        pltpu.sync_copy(ones, degsh.at[colv], add=True)
        return 0
    lax.fori_loop(0, EPT // CH, _edges, 0)

    plsc.subcore_barrier()
    pltpu.sync_copy(degsh.at[pl.ds(sid * NPT, NPT)],
                    out_hbm.at[pl.ds(cid * N + sid * NPT, NPT)])


@functools.partial(
    pl.kernel,
    out_type=jax.ShapeDtypeStruct((2 * N, H), jnp.float32),
    mesh=_mesh,
    scratch_types=[
        pltpu.VMEM((CH,), jnp.int32),
        pltpu.VMEM((CH,), jnp.int32),
        pltpu.VMEM((CH, H), jnp.float32),
        pltpu.VMEM((CH, H), jnp.float32),
        pltpu.VMEM_SHARED((N, H), jnp.float32),
        pltpu.SemaphoreType.DMA,
    ],
)
def _sc_prop(g_hbm, row_hbm, col_hbm, out_hbm, rowv, colv, rows, zb, accsh, sem):
    cid = lax.axis_index("c")
    sid = lax.axis_index("s")

    def _zb(i, _):
        for j in range(H // 16):
            zb[i, pl.ds(j * 16, 16)] = jnp.zeros((16,), jnp.float32)
        return 0
    lax.fori_loop(0, CH, _zb, 0)

    def _zacc(j, _):
        pltpu.sync_copy(zb, accsh.at[pl.ds(sid * NPT + j * CH, CH)])
        return 0
    lax.fori_loop(0, 7, _zacc, 0)
    pltpu.sync_copy(zb.at[pl.ds(0, NPT - 7 * CH)],
                    accsh.at[pl.ds(sid * NPT + 7 * CH, NPT - 7 * CH)])
    plsc.subcore_barrier()

    base = (cid * 16 + sid) * EPT

    def _edges(k, _):
        pltpu.sync_copy(row_hbm.at[pl.ds(base + k * CH, CH)], rowv)
        pltpu.sync_copy(col_hbm.at[pl.ds(base + k * CH, CH)], colv)
        pltpu.async_copy(g_hbm.at[rowv], rows, sem).wait()
        pltpu.sync_copy(rows, accsh.at[colv], add=True)
        return 0
    lax.fori_loop(0, EPT // CH, _edges, 0)

    plsc.subcore_barrier()
    pltpu.sync_copy(accsh.at[pl.ds(sid * NPT, NPT)],
                    out_hbm.at[pl.ds(cid * N + sid * NPT, NPT)])


@functools.partial(
    pl.kernel,
    out_type=(jax.ShapeDtypeStruct((EHP,), jnp.float32),
              jax.ShapeDtypeStruct((EHP,), jnp.int32)),
    mesh=_mesh,
    scratch_types=[
        pltpu.VMEM((CE,), jnp.int32),
        pltpu.VMEM((CE,), jnp.int32),
        pltpu.VMEM((CE, 2 * H), jnp.float32),
        pltpu.VMEM((CE, 2 * H), jnp.float32),
        pltpu.VMEM((CE,), jnp.float32),
        pltpu.VMEM((CE,), jnp.int32),
        pltpu.VMEM((N,), jnp.int32),
        pltpu.VMEM((2 * H,), jnp.float32),
        pltpu.VMEM((8,), jnp.float32),
        pltpu.SemaphoreType.DMA,
        pltpu.SemaphoreType.DMA,
    ],
)
def _sc_edge(ap_hbm, bm_hbm, n0_hbm, n1_hbm, info_hbm, w2_hbm, b2_hbm,
             elog_hbm, ebat_hbm,
             n0v, n1v, bufa, bufb, res, bres, infov, w2v, b2v, sema, semb):
    cid = lax.axis_index("c")
    sid = lax.axis_index("s")
    wid = cid * 16 + sid
    pltpu.sync_copy(info_hbm, infov)
    pltpu.sync_copy(w2_hbm, w2v)
    pltpu.sync_copy(b2_hbm, b2v)
    base = wid * EPTE

    def _chunk(k, _):
        eb = base + k * CE
        pltpu.sync_copy(n0_hbm.at[pl.ds(eb, CE)], n0v)
        pltpu.sync_copy(n1_hbm.at[pl.ds(eb, CE)], n1v)
        ca = pltpu.async_copy(ap_hbm.at[n0v], bufa, sema)
        cb = pltpu.async_copy(bm_hbm.at[n1v], bufb, semb)
        ca.wait()
        cb.wait()
        b2s = b2v[0]
        for g in range(CE // 16):
            lanes = lax.iota(jnp.int32, 16) + g * 16
            n0l = n0v[pl.ds(g * 16, 16)]
            bres[pl.ds(g * 16, 16)] = plsc.load_gather(infov, [n0l])

            def _f(f, acc):
                fs = jnp.full((16,), f, jnp.int32)
                va = plsc.load_gather(bufa, [lanes, fs])
                vb = plsc.load_gather(bufb, [lanes, fs])
                return acc + jnp.maximum(va + vb, 0.0) * w2v[f]
            acc = lax.fori_loop(0, 2 * H, _f, jnp.zeros((16,), jnp.float32))
            res[pl.ds(g * 16, 16)] = acc + b2s
        pltpu.sync_copy(res, elog_hbm.at[pl.ds(eb, CE)])
        pltpu.sync_copy(bres, ebat_hbm.at[pl.ds(eb, CE)])
        return 0
    lax.fori_loop(0, EPTE // CE, _chunk, 0)


BLK = 1000
GRID = N // BLK


def _t1_body(deg0_ref, deg1_ref, x_ref, wt_ref, b_ref, dis_ref, g1_ref):
    cnt = deg0_ref[...][:, 0:1] + deg1_ref[...][:, 0:1]
    dis = lax.rsqrt(cnt + 1.0)
    h = jnp.dot(x_ref[...], wt_ref[...], preferred_element_type=jnp.float32)
    h = h + b_ref[...]
    dis_ref[...] = jnp.broadcast_to(dis, (BLK, 16))
    g1_ref[...] = dis * h


def _t1(deg0, deg1, x, wt, b):
    return pl.pallas_call(
        _t1_body,
        grid=(GRID,),
        in_specs=[
            pl.BlockSpec((BLK, 16), lambda i: (i, 0)),
            pl.BlockSpec((BLK, 16), lambda i: (i, 0)),
            pl.BlockSpec((BLK, H), lambda i: (i, 0)),
            pl.BlockSpec((H, H), lambda i: (0, 0)),
            pl.BlockSpec((1, H), lambda i: (0, 0)),
        ],
        out_specs=[
            pl.BlockSpec((BLK, 16), lambda i: (i, 0)),
            pl.BlockSpec((BLK, H), lambda i: (i, 0)),
        ],
        out_shape=[
            jax.ShapeDtypeStruct((N, 16), jnp.float32),
            jax.ShapeDtypeStruct((N, H), jnp.float32),
        ],
    )(deg0, deg1, x, wt, b)


def _t2_body(a0_ref, a1_ref, g1_ref, dis_ref, bias_ref, wt_ref, b_ref, g2_ref):
    dis = dis_ref[...][:, 0:1]
    h1 = dis * (a0_ref[...] + a1_ref[...] + g1_ref[...]) + bias_ref[...]
    h2 = jnp.dot(h1, wt_ref[...], preferred_element_type=jnp.float32) + b_ref[...]
    g2_ref[...] = dis * h2


def _t2(a0, a1, g1, dis16, bias1, wt, b):
    return pl.pallas_call(
        _t2_body,
        grid=(GRID,),
        in_specs=[
            pl.BlockSpec((BLK, H), lambda i: (i, 0)),
            pl.BlockSpec((BLK, H), lambda i: (i, 0)),
            pl.BlockSpec((BLK, H), lambda i: (i, 0)),
            pl.BlockSpec((BLK, 16), lambda i: (i, 0)),
            pl.BlockSpec((1, H), lambda i: (0, 0)),
            pl.BlockSpec((H, H), lambda i: (0, 0)),
            pl.BlockSpec((1, H), lambda i: (0, 0)),
        ],
        out_specs=pl.BlockSpec((BLK, H), lambda i: (i, 0)),
        out_shape=jax.ShapeDtypeStruct((N, H), jnp.float32),
    )(a0, a1, g1, dis16, bias1, wt, b)


def _t3_body(a0_ref, a1_ref, g2_ref, dis_ref, bias_ref,
             ncw1t_ref, ncb1_ref, ncw2_ref, ncb2_ref,
             w1at_ref, b1e_ref, w1bt_ref,
             nlog_ref, ap_ref, bm_ref):
    dis = dis_ref[...][:, 0:1]
    h = dis * (a0_ref[...] + a1_ref[...] + g2_ref[...]) + bias_ref[...]
    z = jnp.maximum(
        jnp.dot(h, ncw1t_ref[...], preferred_element_type=jnp.float32)
        + ncb1_ref[...], 0.0)
    nlog = jnp.sum(z * ncw2_ref[...], axis=1, keepdims=True) + ncb2_ref[...]
    nlog_ref[...] = jnp.broadcast_to(nlog, (BLK, 16))
    ap_ref[...] = jnp.dot(h, w1at_ref[...],
                          preferred_element_type=jnp.float32) + b1e_ref[...]
    bm_ref[...] = jnp.dot(h, w1bt_ref[...], preferred_element_type=jnp.float32)


def _t3(a0, a1, g2, dis16, bias2, ncw1t, ncb1, ncw2, ncb2, w1at, b1e, w1bt):
    return pl.pallas_call(
        _t3_body,
        grid=(GRID,),
        in_specs=[
            pl.BlockSpec((BLK, H), lambda i: (i, 0)),
            pl.BlockSpec((BLK, H), lambda i: (i, 0)),
            pl.BlockSpec((BLK, H), lambda i: (i, 0)),
            pl.BlockSpec((BLK, 16), lambda i: (i, 0)),
            pl.BlockSpec((1, H), lambda i: (0, 0)),
            pl.BlockSpec((H, H), lambda i: (0, 0)),
            pl.BlockSpec((1, H), lambda i: (0, 0)),
            pl.BlockSpec((1, H), lambda i: (0, 0)),
            pl.BlockSpec((1, 1), lambda i: (0, 0)),
            pl.BlockSpec((H, 2 * H), lambda i: (0, 0)),
            pl.BlockSpec((1, 2 * H), lambda i: (0, 0)),
            pl.BlockSpec((H, 2 * H), lambda i: (0, 0)),
        ],
        out_specs=[
            pl.BlockSpec((BLK, 16), lambda i: (i, 0)),
            pl.BlockSpec((BLK, 2 * H), lambda i: (i, 0)),
            pl.BlockSpec((BLK, 2 * H), lambda i: (i, 0)),
        ],
        out_shape=[
            jax.ShapeDtypeStruct((N, 16), jnp.float32),
            jax.ShapeDtypeStruct((N, 2 * H), jnp.float32),
            jax.ShapeDtypeStruct((N, 2 * H), jnp.float32),
        ],
    )(a0, a1, g2, dis16, bias2, ncw1t, ncb1, ncw2, ncb2, w1at, b1e, w1bt)


CL = N + EH
CLR = 1330
CLP = CLR * 128


def _soft_body(cl_ref, cb_ref, out_ref, msel_ref):
    msel_ref[...] = jnp.full((CLR, 128), 1e30, jnp.float32)

    def _m(b, _):
        cb = cb_ref[...]
        mb = jnp.max(jnp.where(cb == b, cl_ref[...], -1e30))
        msel_ref[...] = jnp.where(cb == b, mb, msel_ref[...])
        return 0
    lax.fori_loop(0, B, _m, 0)

    out_ref[...] = jnp.exp(cl_ref[...] - msel_ref[...])

    def _s(b, _):
        cb = cb_ref[...]
        sb = jnp.sum(jnp.where(cb == b, out_ref[...], 0.0))
        msel_ref[...] = jnp.where(cb == b, 1.0 / sb, msel_ref[...])
        return 0
    lax.fori_loop(0, B, _s, 0)

    out_ref[...] = out_ref[...] * jnp.where(cb_ref[...] < B, msel_ref[...], 0.0)


def _soft(cl2, cb2):
    return pl.pallas_call(
        _soft_body,
        grid=(1,),
        in_specs=[
            pl.BlockSpec((CLR, 128), lambda i: (0, 0)),
            pl.BlockSpec((CLR, 128), lambda i: (0, 0)),
        ],
        out_specs=pl.BlockSpec((CLR, 128), lambda i: (0, 0)),
        out_shape=jax.ShapeDtypeStruct((CLR, 128), jnp.float32),
        scratch_shapes=[pltpu.VMEM((CLR, 128), jnp.float32)],
    )(cl2, cb2)


def kernel(x, edge_index, edge_attr, info_batch,
           W_g1, b_g1, bias_g1, W_g2, b_g2, bias_g2,
           nc_W1, nc_b1, nc_W2, nc_b2, ec_W1, ec_b1, ec_W2, ec_b2):
    row = edge_index[0]
    col = edge_index[1]
    pad = EHP - EH
    n0p = jnp.concatenate([row[::2], jnp.zeros((pad,), jnp.int32)])
    n1p = jnp.concatenate([col[::2], jnp.zeros((pad,), jnp.int32)])

    degout = _sc_deg(col)
    deg0, deg1 = degout[:N], degout[N:]

    dis16, g1 = _t1(deg0, deg1, x, W_g1.T, b_g1[None, :])

    acc1 = _sc_prop(g1, row, col)
    g2 = _t2(acc1[:N], acc1[N:], g1, dis16, bias_g1[None, :],
             W_g2.T, b_g2[None, :])

    acc2 = _sc_prop(g2, row, col)
    nlog16, ap, bm = _t3(acc2[:N], acc2[N:], g2, dis16, bias_g2[None, :],
                         nc_W1.T, nc_b1[None, :], nc_W2, nc_b2[None, None],
                         ec_W1[:, :H].T, ec_b1[None, :], ec_W1[:, H:].T)

    elog, ebat = _sc_edge(ap, bm, n0p, n1p, info_batch, ec_W2[0],
                          jnp.pad(ec_b2, (0, 7)))

    cl = jnp.concatenate([nlog16[:, 0], elog[:EH],
                          jnp.zeros((CLP - CL,), jnp.float32)])
    cb = jnp.concatenate([info_batch, ebat[:EH],
                          jnp.full((CLP - CL,), B, jnp.int32)])
    soft = _soft(cl.reshape(CLR, 128), cb.reshape(CLR, 128)).reshape(-1)
    return (soft[:N, None], soft[N:CL, None])


# speedup vs baseline: 4.4526x; 4.4526x over previous
import functools

import jax
import jax.numpy as jnp
from jax import lax
from jax.experimental import pallas as pl
from jax.experimental.pallas import tpu as pltpu
from jax.experimental.pallas import tpu_sc as plsc

N = 10000
E = 320000
H = 128
B = 64
NW = 32
EPT = E // NW
CH = 80
NP = 10240
NPT = NP // 16
EH = E // 2
EHP = 163840
EPTE = EHP // NW
CE = 128

_mesh = plsc.VectorSubcoreMesh(core_axis_name="c", subcore_axis_name="s")


@functools.partial(
    pl.kernel,
    out_type=jax.ShapeDtypeStruct((2 * NP, H), jnp.float32),
    mesh=_mesh,
    scratch_types=[
        pltpu.VMEM((CH,), jnp.int32),
        pltpu.VMEM((CH, H), jnp.float32),
        pltpu.VMEM((CH, H), jnp.float32),
        pltpu.VMEM_SHARED((NP, H), jnp.float32),
    ],
)
def _sc_deg(col_hbm, out_hbm, colv, ones, zb, degsh):
    cid = lax.axis_index("c")
    sid = lax.axis_index("s")

    def _zb(i, _):
        for j in range(H // 16):
            zb[i, pl.ds(j * 16, 16)] = jnp.zeros((16,), jnp.float32)
            ones[i, pl.ds(j * 16, 16)] = jnp.full((16,), 1.0, jnp.float32)
        return 0
    lax.fori_loop(0, CH, _zb, 0)

    def _zacc(j, _):
        pltpu.sync_copy(zb, degsh.at[pl.ds(sid * NPT + j * CH, CH)])
        return 0
    lax.fori_loop(0, NPT // CH, _zacc, 0)
    plsc.subcore_barrier()

    base = (cid * 16 + sid) * EPT

    def _edges(k, _):
        pltpu.sync_copy(col_hbm.at[pl.ds(base + k * CH, CH)], colv)
        pltpu.sync_copy(ones, degsh.at[colv], add=True)
        return 0
    lax.fori_loop(0, EPT // CH, _edges, 0)

    plsc.subcore_barrier()
    pltpu.sync_copy(degsh.at[pl.ds(sid * NPT, NPT)],
                    out_hbm.at[pl.ds(cid * NP + sid * NPT, NPT)])


@functools.partial(
    pl.kernel,
    out_type=jax.ShapeDtypeStruct((2 * NP, H), jnp.float32),
    mesh=_mesh,
    scratch_types=[
        pltpu.VMEM((CH,), jnp.int32),
        pltpu.VMEM((CH,), jnp.int32),
        pltpu.VMEM((CH, H), jnp.float32),
        pltpu.VMEM((CH, H), jnp.float32),
        pltpu.VMEM_SHARED((NP, H), jnp.float32),
        pltpu.SemaphoreType.DMA,
    ],
)
def _sc_prop(g_hbm, row_hbm, col_hbm, out_hbm, rowv, colv, rows, zb, accsh, sem):
    cid = lax.axis_index("c")
    sid = lax.axis_index("s")

    def _zb(i, _):
        for j in range(H // 16):
            zb[i, pl.ds(j * 16, 16)] = jnp.zeros((16,), jnp.float32)
        return 0
    lax.fori_loop(0, CH, _zb, 0)

    def _zacc(j, _):
        pltpu.sync_copy(zb, accsh.at[pl.ds(sid * NPT + j * CH, CH)])
        return 0
    lax.fori_loop(0, NPT // CH, _zacc, 0)
    plsc.subcore_barrier()

    base = (cid * 16 + sid) * EPT

    def _edges(k, _):
        pltpu.sync_copy(row_hbm.at[pl.ds(base + k * CH, CH)], rowv)
        pltpu.sync_copy(col_hbm.at[pl.ds(base + k * CH, CH)], colv)
        pltpu.async_copy(g_hbm.at[rowv], rows, sem).wait()
        pltpu.sync_copy(rows, accsh.at[colv], add=True)
        return 0
    lax.fori_loop(0, EPT // CH, _edges, 0)

    plsc.subcore_barrier()
    pltpu.sync_copy(accsh.at[pl.ds(sid * NPT, NPT)],
                    out_hbm.at[pl.ds(cid * NP + sid * NPT, NPT)])


@functools.partial(
    pl.kernel,
    out_type=(jax.ShapeDtypeStruct((EHP,), jnp.float32),
              jax.ShapeDtypeStruct((EHP,), jnp.int32)),
    mesh=_mesh,
    scratch_types=[
        pltpu.VMEM((CE,), jnp.int32),
        pltpu.VMEM((CE,), jnp.int32),
        pltpu.VMEM((CE, 2 * H), jnp.float32),
        pltpu.VMEM((CE, 2 * H), jnp.float32),
        pltpu.VMEM((CE,), jnp.float32),
        pltpu.VMEM((CE,), jnp.int32),
        pltpu.VMEM((N,), jnp.int32),
        pltpu.VMEM((2 * H,), jnp.float32),
        pltpu.VMEM((16,), jnp.float32),
        pltpu.SemaphoreType.DMA,
        pltpu.SemaphoreType.DMA,
    ],
    compiler_params=pltpu.CompilerParams(needs_layout_passes=False),
)
def _sc_edge(ap_hbm, bm_hbm, n0_hbm, n1_hbm, info_hbm, w2_hbm, b2_hbm,
             elog_hbm, ebat_hbm,
             n0v, n1v, bufa, bufb, res, bres, infov, w2v, b2v, sema, semb):
    cid = lax.axis_index("c")
    sid = lax.axis_index("s")
    wid = cid * 16 + sid
    pltpu.sync_copy(info_hbm, infov)
    pltpu.sync_copy(w2_hbm, w2v)
    pltpu.sync_copy(b2_hbm, b2v)
    base = wid * EPTE

    def _chunk(k, _):
        eb = base + k * CE
        pltpu.sync_copy(n0_hbm.at[pl.ds(eb, CE)], n0v)
        pltpu.sync_copy(n1_hbm.at[pl.ds(eb, CE)], n1v)
        ca = pltpu.async_copy(ap_hbm.at[n0v], bufa, sema)
        cb = pltpu.async_copy(bm_hbm.at[n1v], bufb, semb)
        ca.wait()
        cb.wait()
        b2s = b2v[pl.ds(0, 16)][0]
        for g in range(CE // 16):
            lanes = lax.iota(jnp.int32, 16) + g * 16
            n0l = n0v[pl.ds(g * 16, 16)]
            bres[pl.ds(g * 16, 16)] = plsc.load_gather(infov, [n0l])

            def _f(fg, acc):
                wvec = w2v[pl.ds(fg * 16, 16)]
                for j in range(16):
                    fs = jnp.full((16,), fg * 16 + j, jnp.int32)
                    va = plsc.load_gather(bufa, [lanes, fs])
                    vb = plsc.load_gather(bufb, [lanes, fs])
                    acc = acc + jnp.maximum(va + vb, 0.0) * wvec[j]
                return acc
            acc = lax.fori_loop(0, (2 * H) // 16, _f,
                                jnp.zeros((16,), jnp.float32))
            res[pl.ds(g * 16, 16)] = acc + b2s
        pltpu.sync_copy(res, elog_hbm.at[pl.ds(eb, CE)])
        pltpu.sync_copy(bres, ebat_hbm.at[pl.ds(eb, CE)])
        return 0
    lax.fori_loop(0, EPTE // CE, _chunk, 0)


BLK = 1000
GRID = N // BLK


def _t1_body(deg0_ref, deg1_ref, x_ref, wt_ref, b_ref, dis_ref, g1_ref):
    cnt = deg0_ref[...][:, 0:1] + deg1_ref[...][:, 0:1]
    dis = lax.rsqrt(cnt + 1.0)
    h = jnp.dot(x_ref[...], wt_ref[...], preferred_element_type=jnp.float32)
    h = h + b_ref[...]
    dis_ref[...] = jnp.broadcast_to(dis, (BLK, 16))
    g1_ref[...] = dis * h


def _t1(deg0, deg1, x, wt, b):
    return pl.pallas_call(
        _t1_body,
        grid=(GRID,),
        in_specs=[
            pl.BlockSpec((BLK, H), lambda i: (i, 0)),
            pl.BlockSpec((BLK, H), lambda i: (i, 0)),
            pl.BlockSpec((BLK, H), lambda i: (i, 0)),
            pl.BlockSpec((H, H), lambda i: (0, 0)),
            pl.BlockSpec((1, H), lambda i: (0, 0)),
        ],
        out_specs=[
            pl.BlockSpec((BLK, 16), lambda i: (i, 0)),
            pl.BlockSpec((BLK, H), lambda i: (i, 0)),
        ],
        out_shape=[
            jax.ShapeDtypeStruct((N, 16), jnp.float32),
            jax.ShapeDtypeStruct((N, H), jnp.float32),
        ],
    )(deg0, deg1, x, wt, b)


def _t2_body(a0_ref, a1_ref, g1_ref, dis_ref, bias_ref, wt_ref, b_ref, g2_ref):
    dis = dis_ref[...][:, 0:1]
    h1 = dis * (a0_ref[...] + a1_ref[...] + g1_ref[...]) + bias_ref[...]
    h2 = jnp.dot(h1, wt_ref[...], preferred_element_type=jnp.float32) + b_ref[...]
    g2_ref[...] = dis * h2


def _t2(a0, a1, g1, dis16, bias1, wt, b):
    return pl.pallas_call(
        _t2_body,
        grid=(GRID,),
        in_specs=[
            pl.BlockSpec((BLK, H), lambda i: (i, 0)),
            pl.BlockSpec((BLK, H), lambda i: (i, 0)),
            pl.BlockSpec((BLK, H), lambda i: (i, 0)),
            pl.BlockSpec((BLK, 16), lambda i: (i, 0)),
            pl.BlockSpec((1, H), lambda i: (0, 0)),
            pl.BlockSpec((H, H), lambda i: (0, 0)),
            pl.BlockSpec((1, H), lambda i: (0, 0)),
        ],
        out_specs=pl.BlockSpec((BLK, H), lambda i: (i, 0)),
        out_shape=jax.ShapeDtypeStruct((N, H), jnp.float32),
    )(a0, a1, g1, dis16, bias1, wt, b)


def _t3_body(a0_ref, a1_ref, g2_ref, dis_ref, bias_ref,
             ncw1t_ref, ncb1_ref, ncw2_ref, ncb2_ref,
             w1at_ref, b1e_ref, w1bt_ref,
             nlog_ref, ap_ref, bm_ref):
    dis = dis_ref[...][:, 0:1]
    h = dis * (a0_ref[...] + a1_ref[...] + g2_ref[...]) + bias_ref[...]
    z = jnp.maximum(
        jnp.dot(h, ncw1t_ref[...], preferred_element_type=jnp.float32)
        + ncb1_ref[...], 0.0)
    nlog = jnp.sum(z * ncw2_ref[...], axis=1, keepdims=True) + ncb2_ref[...]
    nlog_ref[...] = jnp.broadcast_to(nlog, (BLK, 16))
    ap_ref[...] = jnp.dot(h, w1at_ref[...],
                          preferred_element_type=jnp.float32) + b1e_ref[...]
    bm_ref[...] = jnp.dot(h, w1bt_ref[...], preferred_element_type=jnp.float32)


def _t3(a0, a1, g2, dis16, bias2, ncw1t, ncb1, ncw2, ncb2, w1at, b1e, w1bt):
    return pl.pallas_call(
        _t3_body,
        grid=(GRID,),
        in_specs=[
            pl.BlockSpec((BLK, H), lambda i: (i, 0)),
            pl.BlockSpec((BLK, H), lambda i: (i, 0)),
            pl.BlockSpec((BLK, H), lambda i: (i, 0)),
            pl.BlockSpec((BLK, 16), lambda i: (i, 0)),
            pl.BlockSpec((1, H), lambda i: (0, 0)),
            pl.BlockSpec((H, H), lambda i: (0, 0)),
            pl.BlockSpec((1, H), lambda i: (0, 0)),
            pl.BlockSpec((1, H), lambda i: (0, 0)),
            pl.BlockSpec((1, 1), lambda i: (0, 0)),
            pl.BlockSpec((H, 2 * H), lambda i: (0, 0)),
            pl.BlockSpec((1, 2 * H), lambda i: (0, 0)),
            pl.BlockSpec((H, 2 * H), lambda i: (0, 0)),
        ],
        out_specs=[
            pl.BlockSpec((BLK, 16), lambda i: (i, 0)),
            pl.BlockSpec((BLK, 2 * H), lambda i: (i, 0)),
            pl.BlockSpec((BLK, 2 * H), lambda i: (i, 0)),
        ],
        out_shape=[
            jax.ShapeDtypeStruct((N, 16), jnp.float32),
            jax.ShapeDtypeStruct((N, 2 * H), jnp.float32),
            jax.ShapeDtypeStruct((N, 2 * H), jnp.float32),
        ],
    )(a0, a1, g2, dis16, bias2, ncw1t, ncb1, ncw2, ncb2, w1at, b1e, w1bt)


CL = N + EH
CLR = 1330
CLP = CLR * 128


def _soft_body(cl_ref, cb_ref, out_ref, msel_ref):
    msel_ref[...] = jnp.full((CLR, 128), 1e30, jnp.float32)

    def _m(b, _):
        cb = cb_ref[...]
        mb = jnp.max(jnp.where(cb == b, cl_ref[...], -1e30))
        msel_ref[...] = jnp.where(cb == b, mb, msel_ref[...])
        return 0
    lax.fori_loop(0, B, _m, 0)

    out_ref[...] = jnp.exp(cl_ref[...] - msel_ref[...])

    def _s(b, _):
        cb = cb_ref[...]
        sb = jnp.sum(jnp.where(cb == b, out_ref[...], 0.0))
        msel_ref[...] = jnp.where(cb == b, 1.0 / sb, msel_ref[...])
        return 0
    lax.fori_loop(0, B, _s, 0)

    out_ref[...] = out_ref[...] * jnp.where(cb_ref[...] < B, msel_ref[...], 0.0)


def _soft(cl2, cb2):
    return pl.pallas_call(
        _soft_body,
        grid=(1,),
        in_specs=[
            pl.BlockSpec((CLR, 128), lambda i: (0, 0)),
            pl.BlockSpec((CLR, 128), lambda i: (0, 0)),
        ],
        out_specs=pl.BlockSpec((CLR, 128), lambda i: (0, 0)),
        out_shape=jax.ShapeDtypeStruct((CLR, 128), jnp.float32),
        scratch_shapes=[pltpu.VMEM((CLR, 128), jnp.float32)],
    )(cl2, cb2)


def kernel(x, edge_index, edge_attr, info_batch,
           W_g1, b_g1, bias_g1, W_g2, b_g2, bias_g2,
           nc_W1, nc_b1, nc_W2, nc_b2, ec_W1, ec_b1, ec_W2, ec_b2):
    row = edge_index[0]
    col = edge_index[1]
    pad = EHP - EH
    n0p = jnp.concatenate([row[::2], jnp.zeros((pad,), jnp.int32)])
    n1p = jnp.concatenate([col[::2], jnp.zeros((pad,), jnp.int32)])

    degout = _sc_deg(col)
    deg0, deg1 = degout[:N], degout[NP:NP + N]

    dis16, g1 = _t1(deg0, deg1, x, W_g1.T, b_g1[None, :])

    acc1 = _sc_prop(g1, row, col)
    g2 = _t2(acc1[:N], acc1[NP:NP + N], g1, dis16, bias_g1[None, :],
             W_g2.T, b_g2[None, :])

    acc2 = _sc_prop(g2, row, col)
    nlog16, ap, bm = _t3(acc2[:N], acc2[NP:NP + N], g2, dis16, bias_g2[None, :],
                         nc_W1.T, nc_b1[None, :], nc_W2, nc_b2[None, :],
                         ec_W1[:, :H].T, ec_b1[None, :], ec_W1[:, H:].T)

    elog, ebat = _sc_edge(ap, bm, n0p, n1p, info_batch, ec_W2[0],
                          jnp.pad(ec_b2, (0, 15)))

    cl = jnp.concatenate([nlog16[:, 0], elog[:EH],
                          jnp.zeros((CLP - CL,), jnp.float32)])
    cb = jnp.concatenate([info_batch, ebat[:EH],
                          jnp.full((CLP - CL,), B, jnp.int32)])
    soft = _soft(cl.reshape(CLR, 128), cb.reshape(CLR, 128)).reshape(-1)
    return (soft[:N, None], soft[N:CL, None])


# speedup vs baseline: 7.9120x; 1.7770x over previous
import functools

import jax
import jax.numpy as jnp
from jax import lax
from jax.experimental import pallas as pl
from jax.experimental.pallas import tpu as pltpu
from jax.experimental.pallas import tpu_sc as plsc

N = 10000
E = 320000
H = 128
B = 64
NW = 32
EPT = E // NW
CH = 80
NP = 10240
NPT = NP // 16
EH = E // 2
EHP = 163840
EPTE = EHP // NW
CE = 128

_mesh = plsc.VectorSubcoreMesh(core_axis_name="c", subcore_axis_name="s")


@functools.partial(
    pl.kernel,
    out_type=jax.ShapeDtypeStruct((2 * NP, H), jnp.float32),
    mesh=_mesh,
    scratch_types=[
        pltpu.VMEM((CH,), jnp.int32),
        pltpu.VMEM((CH, H), jnp.float32),
        pltpu.VMEM((CH, H), jnp.float32),
        pltpu.VMEM_SHARED((NP, H), jnp.float32),
    ],
)
def _sc_deg(col_hbm, out_hbm, colv, ones, zb, degsh):
    cid = lax.axis_index("c")
    sid = lax.axis_index("s")

    def _zb(i, _):
        for j in range(H // 16):
            zb[i, pl.ds(j * 16, 16)] = jnp.zeros((16,), jnp.float32)
            ones[i, pl.ds(j * 16, 16)] = jnp.full((16,), 1.0, jnp.float32)
        return 0
    lax.fori_loop(0, CH, _zb, 0)

    def _zacc(j, _):
        pltpu.sync_copy(zb, degsh.at[pl.ds(sid * NPT + j * CH, CH)])
        return 0
    lax.fori_loop(0, NPT // CH, _zacc, 0)
    plsc.subcore_barrier()

    base = (cid * 16 + sid) * EPT

    def _edges(k, _):
        pltpu.sync_copy(col_hbm.at[pl.ds(base + k * CH, CH)], colv)
        pltpu.sync_copy(ones, degsh.at[colv], add=True)
        return 0
    lax.fori_loop(0, EPT // CH, _edges, 0)

    plsc.subcore_barrier()
    pltpu.sync_copy(degsh.at[pl.ds(sid * NPT, NPT)],
                    out_hbm.at[pl.ds(cid * NP + sid * NPT, NPT)])


@functools.partial(
    pl.kernel,
    out_type=jax.ShapeDtypeStruct((2 * NP, H), jnp.float32),
    mesh=_mesh,
    scratch_types=[
        pltpu.VMEM((CH,), jnp.int32),
        pltpu.VMEM((CH,), jnp.int32),
        pltpu.VMEM((CH, H), jnp.float32),
        pltpu.VMEM((CH, H), jnp.float32),
        pltpu.VMEM_SHARED((NP, H), jnp.float32),
        pltpu.SemaphoreType.DMA,
    ],
)
def _sc_prop(g_hbm, row_hbm, col_hbm, out_hbm, rowv, colv, rows, zb, accsh, sem):
    cid = lax.axis_index("c")
    sid = lax.axis_index("s")

    def _zb(i, _):
        for j in range(H // 16):
            zb[i, pl.ds(j * 16, 16)] = jnp.zeros((16,), jnp.float32)
        return 0
    lax.fori_loop(0, CH, _zb, 0)

    def _zacc(j, _):
        pltpu.sync_copy(zb, accsh.at[pl.ds(sid * NPT + j * CH, CH)])
        return 0
    lax.fori_loop(0, NPT // CH, _zacc, 0)
    plsc.subcore_barrier()

    base = (cid * 16 + sid) * EPT

    def _edges(k, _):
        pltpu.sync_copy(row_hbm.at[pl.ds(base + k * CH, CH)], rowv)
        pltpu.sync_copy(col_hbm.at[pl.ds(base + k * CH, CH)], colv)
        pltpu.async_copy(g_hbm.at[rowv], rows, sem).wait()
        pltpu.sync_copy(rows, accsh.at[colv], add=True)
        return 0
    lax.fori_loop(0, EPT // CH, _edges, 0)

    plsc.subcore_barrier()
    pltpu.sync_copy(accsh.at[pl.ds(sid * NPT, NPT)],
                    out_hbm.at[pl.ds(cid * NP + sid * NPT, NPT)])


@functools.partial(
    pl.kernel,
    out_type=(jax.ShapeDtypeStruct((EHP,), jnp.float32),
              jax.ShapeDtypeStruct((EHP,), jnp.int32)),
    mesh=_mesh,
    scratch_types=[
        pltpu.VMEM((CE,), jnp.int32),
        pltpu.VMEM((CE,), jnp.int32),
        pltpu.VMEM((CE, 2 * H), jnp.float32),
        pltpu.VMEM((CE, 2 * H), jnp.float32),
        pltpu.VMEM((EPTE,), jnp.float32),
        pltpu.VMEM((EPTE,), jnp.int32),
        pltpu.VMEM((N,), jnp.int32),
        pltpu.VMEM((2 * H,), jnp.float32),
        pltpu.VMEM((16,), jnp.float32),
        pltpu.SemaphoreType.DMA,
        pltpu.SemaphoreType.DMA,
    ],
    compiler_params=pltpu.CompilerParams(needs_layout_passes=False),
)
def _sc_edge(ap_hbm, bm_hbm, n0_hbm, n1_hbm, info_hbm, w2_hbm, b2_hbm,
             elog_hbm, ebat_hbm,
             n0v, n1v, bufa, bufb, resbig, bresbig, infov, w2v, b2v,
             sema, semb):
    cid = lax.axis_index("c")
    sid = lax.axis_index("s")
    wid = cid * 16 + sid
    pltpu.sync_copy(info_hbm, infov)
    pltpu.sync_copy(w2_hbm, w2v)
    pltpu.sync_copy(b2_hbm, b2v)
    base = wid * EPTE
    w2s = [w2v[pl.ds(j * 16, 16)] for j in range(2 * H // 16)]
    b2s = b2v[pl.ds(0, 16)][0]

    def _chunk(k, _):
        eb = base + k * CE
        pltpu.sync_copy(n0_hbm.at[pl.ds(eb, CE)], n0v)
        pltpu.sync_copy(n1_hbm.at[pl.ds(eb, CE)], n1v)
        ca = pltpu.async_copy(ap_hbm.at[n0v], bufa, sema)
        cb = pltpu.async_copy(bm_hbm.at[n1v], bufb, semb)
        ca.wait()
        cb.wait()

        def _grp(g, _):
            n0l = n0v[pl.ds(g * 16, 16)]
            bresbig[pl.ds(k * CE + g * 16, 16)] = plsc.load_gather(
                infov, [n0l])
            rv = jnp.zeros((16,), jnp.float32)
            for e16 in range(16):
                e = g * 16 + e16
                acc = jnp.zeros((16,), jnp.float32)
                for j in range(2 * H // 16):
                    av = bufa[e, pl.ds(j * 16, 16)]
                    bv = bufb[e, pl.ds(j * 16, 16)]
                    acc = acc + jnp.maximum(av + bv, 0.0) * w2s[j]
                sc = jnp.sum(acc)
                rv = jnp.where(lax.iota(jnp.int32, 16) == e16, sc, rv)
            resbig[pl.ds(k * CE + g * 16, 16)] = rv + b2s
            return 0
        lax.fori_loop(0, CE // 16, _grp, 0)
        return 0
    lax.fori_loop(0, EPTE // CE, _chunk, 0)
    pltpu.sync_copy(resbig, elog_hbm.at[pl.ds(base, EPTE)])
    pltpu.sync_copy(bresbig, ebat_hbm.at[pl.ds(base, EPTE)])


BLK = 1000
GRID = N // BLK


def _t1_body(deg0_ref, deg1_ref, x_ref, wt_ref, b_ref, dis_ref, g1_ref):
    cnt = deg0_ref[...][:, 0:1] + deg1_ref[...][:, 0:1]
    dis = lax.rsqrt(cnt + 1.0)
    h = jnp.dot(x_ref[...], wt_ref[...], preferred_element_type=jnp.float32)
    h = h + b_ref[...]
    dis_ref[...] = jnp.broadcast_to(dis, (BLK, 16))
    g1_ref[...] = dis * h


def _t1(deg0, deg1, x, wt, b):
    return pl.pallas_call(
        _t1_body,
        grid=(GRID,),
        in_specs=[
            pl.BlockSpec((BLK, H), lambda i: (i, 0)),
            pl.BlockSpec((BLK, H), lambda i: (i, 0)),
            pl.BlockSpec((BLK, H), lambda i: (i, 0)),
            pl.BlockSpec((H, H), lambda i: (0, 0)),
            pl.BlockSpec((1, H), lambda i: (0, 0)),
        ],
        out_specs=[
            pl.BlockSpec((BLK, 16), lambda i: (i, 0)),
            pl.BlockSpec((BLK, H), lambda i: (i, 0)),
        ],
        out_shape=[
            jax.ShapeDtypeStruct((N, 16), jnp.float32),
            jax.ShapeDtypeStruct((N, H), jnp.float32),
        ],
    )(deg0, deg1, x, wt, b)


def _t2_body(a0_ref, a1_ref, g1_ref, dis_ref, bias_ref, wt_ref, b_ref, g2_ref):
    dis = dis_ref[...][:, 0:1]
    h1 = dis * (a0_ref[...] + a1_ref[...] + g1_ref[...]) + bias_ref[...]
    h2 = jnp.dot(h1, wt_ref[...], preferred_element_type=jnp.float32) + b_ref[...]
    g2_ref[...] = dis * h2


def _t2(a0, a1, g1, dis16, bias1, wt, b):
    return pl.pallas_call(
        _t2_body,
        grid=(GRID,),
        in_specs=[
            pl.BlockSpec((BLK, H), lambda i: (i, 0)),
            pl.BlockSpec((BLK, H), lambda i: (i, 0)),
            pl.BlockSpec((BLK, H), lambda i: (i, 0)),
            pl.BlockSpec((BLK, 16), lambda i: (i, 0)),
            pl.BlockSpec((1, H), lambda i: (0, 0)),
            pl.BlockSpec((H, H), lambda i: (0, 0)),
            pl.BlockSpec((1, H), lambda i: (0, 0)),
        ],
        out_specs=pl.BlockSpec((BLK, H), lambda i: (i, 0)),
        out_shape=jax.ShapeDtypeStruct((N, H), jnp.float32),
    )(a0, a1, g1, dis16, bias1, wt, b)


def _t3_body(a0_ref, a1_ref, g2_ref, dis_ref, bias_ref,
             ncw1t_ref, ncb1_ref, ncw2_ref, ncb2_ref,
             w1at_ref, b1e_ref, w1bt_ref,
             nlog_ref, ap_ref, bm_ref):
    dis = dis_ref[...][:, 0:1]
    h = dis * (a0_ref[...] + a1_ref[...] + g2_ref[...]) + bias_ref[...]
    z = jnp.maximum(
        jnp.dot(h, ncw1t_ref[...], preferred_element_type=jnp.float32)
        + ncb1_ref[...], 0.0)
    nlog = jnp.sum(z * ncw2_ref[...], axis=1, keepdims=True) + ncb2_ref[...]
    nlog_ref[...] = jnp.broadcast_to(nlog, (BLK, 16))
    ap_ref[...] = jnp.dot(h, w1at_ref[...],
                          preferred_element_type=jnp.float32) + b1e_ref[...]
    bm_ref[...] = jnp.dot(h, w1bt_ref[...], preferred_element_type=jnp.float32)


def _t3(a0, a1, g2, dis16, bias2, ncw1t, ncb1, ncw2, ncb2, w1at, b1e, w1bt):
    return pl.pallas_call(
        _t3_body,
        grid=(GRID,),
        in_specs=[
            pl.BlockSpec((BLK, H), lambda i: (i, 0)),
            pl.BlockSpec((BLK, H), lambda i: (i, 0)),
            pl.BlockSpec((BLK, H), lambda i: (i, 0)),
            pl.BlockSpec((BLK, 16), lambda i: (i, 0)),
            pl.BlockSpec((1, H), lambda i: (0, 0)),
            pl.BlockSpec((H, H), lambda i: (0, 0)),
            pl.BlockSpec((1, H), lambda i: (0, 0)),
            pl.BlockSpec((1, H), lambda i: (0, 0)),
            pl.BlockSpec((1, 1), lambda i: (0, 0)),
            pl.BlockSpec((H, 2 * H), lambda i: (0, 0)),
            pl.BlockSpec((1, 2 * H), lambda i: (0, 0)),
            pl.BlockSpec((H, 2 * H), lambda i: (0, 0)),
        ],
        out_specs=[
            pl.BlockSpec((BLK, 16), lambda i: (i, 0)),
            pl.BlockSpec((BLK, 2 * H), lambda i: (i, 0)),
            pl.BlockSpec((BLK, 2 * H), lambda i: (i, 0)),
        ],
        out_shape=[
            jax.ShapeDtypeStruct((N, 16), jnp.float32),
            jax.ShapeDtypeStruct((N, 2 * H), jnp.float32),
            jax.ShapeDtypeStruct((N, 2 * H), jnp.float32),
        ],
    )(a0, a1, g2, dis16, bias2, ncw1t, ncb1, ncw2, ncb2, w1at, b1e, w1bt)


CL = N + EH
CLR = 1330
CLP = CLR * 128


def _soft_body(cl_ref, cb_ref, out_ref, msel_ref):
    msel_ref[...] = jnp.full((CLR, 128), 1e30, jnp.float32)

    def _m(b, _):
        cb = cb_ref[...]
        mb = jnp.max(jnp.where(cb == b, cl_ref[...], -1e30))
        msel_ref[...] = jnp.where(cb == b, mb, msel_ref[...])
        return 0
    lax.fori_loop(0, B, _m, 0)

    out_ref[...] = jnp.exp(cl_ref[...] - msel_ref[...])

    def _s(b, _):
        cb = cb_ref[...]
        sb = jnp.sum(jnp.where(cb == b, out_ref[...], 0.0))
        msel_ref[...] = jnp.where(cb == b, 1.0 / sb, msel_ref[...])
        return 0
    lax.fori_loop(0, B, _s, 0)

    out_ref[...] = out_ref[...] * jnp.where(cb_ref[...] < B, msel_ref[...], 0.0)


def _soft(cl2, cb2):
    return pl.pallas_call(
        _soft_body,
        grid=(1,),
        in_specs=[
            pl.BlockSpec((CLR, 128), lambda i: (0, 0)),
            pl.BlockSpec((CLR, 128), lambda i: (0, 0)),
        ],
        out_specs=pl.BlockSpec((CLR, 128), lambda i: (0, 0)),
        out_shape=jax.ShapeDtypeStruct((CLR, 128), jnp.float32),
        scratch_shapes=[pltpu.VMEM((CLR, 128), jnp.float32)],
    )(cl2, cb2)


def kernel(x, edge_index, edge_attr, info_batch,
           W_g1, b_g1, bias_g1, W_g2, b_g2, bias_g2,
           nc_W1, nc_b1, nc_W2, nc_b2, ec_W1, ec_b1, ec_W2, ec_b2):
    row = edge_index[0]
    col = edge_index[1]
    pad = EHP - EH
    n0p = jnp.concatenate([row[::2], jnp.zeros((pad,), jnp.int32)])
    n1p = jnp.concatenate([col[::2], jnp.zeros((pad,), jnp.int32)])

    degout = _sc_deg(col)
    deg0, deg1 = degout[:N], degout[NP:NP + N]

    dis16, g1 = _t1(deg0, deg1, x, W_g1.T, b_g1[None, :])

    acc1 = _sc_prop(g1, row, col)
    g2 = _t2(acc1[:N], acc1[NP:NP + N], g1, dis16, bias_g1[None, :],
             W_g2.T, b_g2[None, :])

    acc2 = _sc_prop(g2, row, col)
    nlog16, ap, bm = _t3(acc2[:N], acc2[NP:NP + N], g2, dis16, bias_g2[None, :],
                         nc_W1.T, nc_b1[None, :], nc_W2, nc_b2[None, :],
                         ec_W1[:, :H].T, ec_b1[None, :], ec_W1[:, H:].T)

    elog, ebat = _sc_edge(ap, bm, n0p, n1p, info_batch, ec_W2[0],
                          jnp.pad(ec_b2, (0, 15)))

    cl = jnp.concatenate([nlog16[:, 0], elog[:EH],
                          jnp.zeros((CLP - CL,), jnp.float32)])
    cb = jnp.concatenate([info_batch, ebat[:EH],
                          jnp.full((CLP - CL,), B, jnp.int32)])
    soft = _soft(cl.reshape(CLR, 128), cb.reshape(CLR, 128)).reshape(-1)
    return (soft[:N, None], soft[N:CL, None])


# speedup vs baseline: 9.1335x; 1.1544x over previous
import functools

import jax
import jax.numpy as jnp
from jax import lax
from jax.experimental import pallas as pl
from jax.experimental.pallas import tpu as pltpu
from jax.experimental.pallas import tpu_sc as plsc

N = 10000
E = 320000
H = 128
B = 64
NW = 32
EPT = E // NW
CH = 80
NP = 10240
NPT = NP // 16
EH = E // 2
EHP = 163840
EPTE = EHP // NW
CE = 128
CE2 = 64

_mesh = plsc.VectorSubcoreMesh(core_axis_name="c", subcore_axis_name="s")


@functools.partial(
    pl.kernel,
    out_type=jax.ShapeDtypeStruct((2 * NP, H), jnp.float32),
    mesh=_mesh,
    scratch_types=[
        pltpu.VMEM((CH,), jnp.int32),
        pltpu.VMEM((CH, H), jnp.float32),
        pltpu.VMEM((CH, H), jnp.float32),
        pltpu.VMEM_SHARED((NP, H), jnp.float32),
    ],
)
def _sc_deg(col_hbm, out_hbm, colv, ones, zb, degsh):
    cid = lax.axis_index("c")
    sid = lax.axis_index("s")

    def _zb(i, _):
        for j in range(H // 16):
            zb[i, pl.ds(j * 16, 16)] = jnp.zeros((16,), jnp.float32)
            ones[i, pl.ds(j * 16, 16)] = jnp.full((16,), 1.0, jnp.float32)
        return 0
    lax.fori_loop(0, CH, _zb, 0)

    def _zacc(j, _):
        pltpu.sync_copy(zb, degsh.at[pl.ds(sid * NPT + j * CH, CH)])
        return 0
    lax.fori_loop(0, NPT // CH, _zacc, 0)
    plsc.subcore_barrier()

    base = (cid * 16 + sid) * EPT

    def _edges(k, _):
        pltpu.sync_copy(col_hbm.at[pl.ds(base + k * CH, CH)], colv)
        pltpu.sync_copy(ones, degsh.at[colv], add=True)
        return 0
    lax.fori_loop(0, EPT // CH, _edges, 0)

    plsc.subcore_barrier()
    pltpu.sync_copy(degsh.at[pl.ds(sid * NPT, NPT)],
                    out_hbm.at[pl.ds(cid * NP + sid * NPT, NPT)])


@functools.partial(
    pl.kernel,
    out_type=jax.ShapeDtypeStruct((2 * NP, H), jnp.float32),
    mesh=_mesh,
    scratch_types=[
        pltpu.VMEM((CH,), jnp.int32),
        pltpu.VMEM((CH,), jnp.int32),
        pltpu.VMEM((CH, H), jnp.float32),
        pltpu.VMEM((CH, H), jnp.float32),
        pltpu.VMEM_SHARED((NP, H), jnp.float32),
        pltpu.SemaphoreType.DMA,
    ],
)
def _sc_prop(g_hbm, row_hbm, col_hbm, out_hbm, rowv, colv, rows, zb, accsh, sem):
    cid = lax.axis_index("c")
    sid = lax.axis_index("s")

    def _zb(i, _):
        for j in range(H // 16):
            zb[i, pl.ds(j * 16, 16)] = jnp.zeros((16,), jnp.float32)
        return 0
    lax.fori_loop(0, CH, _zb, 0)

    def _zacc(j, _):
        pltpu.sync_copy(zb, accsh.at[pl.ds(sid * NPT + j * CH, CH)])
        return 0
    lax.fori_loop(0, NPT // CH, _zacc, 0)
    plsc.subcore_barrier()

    base = (cid * 16 + sid) * EPT

    def _edges(k, _):
        pltpu.sync_copy(row_hbm.at[pl.ds(base + k * CH, CH)], rowv)
        pltpu.sync_copy(col_hbm.at[pl.ds(base + k * CH, CH)], colv)
        pltpu.async_copy(g_hbm.at[rowv], rows, sem).wait()
        pltpu.sync_copy(rows, accsh.at[colv], add=True)
        return 0
    lax.fori_loop(0, EPT // CH, _edges, 0)

    plsc.subcore_barrier()
    pltpu.sync_copy(accsh.at[pl.ds(sid * NPT, NPT)],
                    out_hbm.at[pl.ds(cid * NP + sid * NPT, NPT)])


@functools.partial(
    pl.kernel,
    out_type=(jax.ShapeDtypeStruct((EHP,), jnp.float32),
              jax.ShapeDtypeStruct((EHP,), jnp.int32)),
    mesh=_mesh,
    scratch_types=[
        pltpu.VMEM((CE2,), jnp.int32),
        pltpu.VMEM((CE2,), jnp.int32),
        pltpu.VMEM((CE2,), jnp.int32),
        pltpu.VMEM((CE2,), jnp.int32),
        pltpu.VMEM((CE2, 2 * H), jnp.float32),
        pltpu.VMEM((CE2, 2 * H), jnp.float32),
        pltpu.VMEM((CE2, 2 * H), jnp.float32),
        pltpu.VMEM((CE2, 2 * H), jnp.float32),
        pltpu.VMEM((EPTE,), jnp.float32),
        pltpu.VMEM((EPTE,), jnp.int32),
        pltpu.VMEM((N,), jnp.int32),
        pltpu.VMEM((2 * H,), jnp.float32),
        pltpu.VMEM((16,), jnp.float32),
        pltpu.SemaphoreType.DMA,
        pltpu.SemaphoreType.DMA,
        pltpu.SemaphoreType.DMA,
        pltpu.SemaphoreType.DMA,
    ],
    compiler_params=pltpu.CompilerParams(needs_layout_passes=False),
)
def _sc_edge(ap_hbm, bm_hbm, n0_hbm, n1_hbm, info_hbm, w2_hbm, b2_hbm,
             elog_hbm, ebat_hbm,
             n0v0, n0v1, n1v0, n1v1, bufa0, bufa1, bufb0, bufb1,
             resbig, bresbig, infov, w2v, b2v,
             sema0, sema1, semb0, semb1):
    cid = lax.axis_index("c")
    sid = lax.axis_index("s")
    wid = cid * 16 + sid
    pltpu.sync_copy(info_hbm, infov)
    pltpu.sync_copy(w2_hbm, w2v)
    pltpu.sync_copy(b2_hbm, b2v)
    base = wid * EPTE
    w2s = [w2v[pl.ds(j * 16, 16)] for j in range(2 * H // 16)]
    b2s = b2v[pl.ds(0, 16)][0]
    NCH = EPTE // CE2
    slots = ((n0v0, n1v0, bufa0, bufb0, sema0, semb0),
             (n0v1, n1v1, bufa1, bufb1, sema1, semb1))

    def _fire(k, sl):
        n0v, n1v, bufa, bufb, sema, semb = sl
        eb = base + k * CE2
        pltpu.sync_copy(n0_hbm.at[pl.ds(eb, CE2)], n0v)
        pltpu.sync_copy(n1_hbm.at[pl.ds(eb, CE2)], n1v)
        pltpu.async_copy(ap_hbm.at[n0v], bufa, sema)
        pltpu.async_copy(bm_hbm.at[n1v], bufb, semb)

    for b in (0, 1):
        _fire(b, slots[b])

    def _pair(i, _):
        for b in (0, 1):
            k = 2 * i + b
            n0v, n1v, bufa, bufb, sema, semb = slots[b]
            pltpu.make_async_copy(ap_hbm.at[n0v], bufa, sema).wait()
            pltpu.make_async_copy(bm_hbm.at[n1v], bufb, semb).wait()

            def _grp(g, _):
                n0l = n0v[pl.ds(g * 16, 16)]
                bresbig[pl.ds(k * CE2 + g * 16, 16)] = plsc.load_gather(
                    infov, [n0l])
                rv = jnp.zeros((16,), jnp.float32)
                for e16 in range(16):
                    e = g * 16 + e16
                    acc = jnp.zeros((16,), jnp.float32)
                    for j in range(2 * H // 16):
                        av = bufa[e, pl.ds(j * 16, 16)]
                        bv = bufb[e, pl.ds(j * 16, 16)]
                        acc = acc + jnp.maximum(av + bv, 0.0) * w2s[j]
                    sc = jnp.sum(acc)
                    rv = jnp.where(lax.iota(jnp.int32, 16) == e16, sc, rv)
                resbig[pl.ds(k * CE2 + g * 16, 16)] = rv + b2s
                return 0
            lax.fori_loop(0, CE2 // 16, _grp, 0)

            @pl.when(k + 2 < NCH)
            def _():
                _fire(k + 2, slots[b])
        return 0
    lax.fori_loop(0, NCH // 2, _pair, 0)
    pltpu.sync_copy(resbig, elog_hbm.at[pl.ds(base, EPTE)])
    pltpu.sync_copy(bresbig, ebat_hbm.at[pl.ds(base, EPTE)])


BLK = 1000
GRID = N // BLK


def _t1_body(deg0_ref, deg1_ref, x_ref, wt_ref, b_ref, dis_ref, g1_ref):
    cnt = deg0_ref[...][:, 0:1] + deg1_ref[...][:, 0:1]
    dis = lax.rsqrt(cnt + 1.0)
    h = jnp.dot(x_ref[...], wt_ref[...], preferred_element_type=jnp.float32)
    h = h + b_ref[...]
    dis_ref[...] = jnp.broadcast_to(dis, (BLK, 16))
    g1_ref[...] = dis * h


def _t1(deg0, deg1, x, wt, b):
    return pl.pallas_call(
        _t1_body,
        grid=(GRID,),
        in_specs=[
            pl.BlockSpec((BLK, H), lambda i: (i, 0)),
            pl.BlockSpec((BLK, H), lambda i: (i, 0)),
            pl.BlockSpec((BLK, H), lambda i: (i, 0)),
            pl.BlockSpec((H, H), lambda i: (0, 0)),
            pl.BlockSpec((1, H), lambda i: (0, 0)),
        ],
        out_specs=[
            pl.BlockSpec((BLK, 16), lambda i: (i, 0)),
            pl.BlockSpec((BLK, H), lambda i: (i, 0)),
        ],
        out_shape=[
            jax.ShapeDtypeStruct((N, 16), jnp.float32),
            jax.ShapeDtypeStruct((N, H), jnp.float32),
        ],
    )(deg0, deg1, x, wt, b)


def _t2_body(a0_ref, a1_ref, g1_ref, dis_ref, bias_ref, wt_ref, b_ref, g2_ref):
    dis = dis_ref[...][:, 0:1]
    h1 = dis * (a0_ref[...] + a1_ref[...] + g1_ref[...]) + bias_ref[...]
    h2 = jnp.dot(h1, wt_ref[...], preferred_element_type=jnp.float32) + b_ref[...]
    g2_ref[...] = dis * h2


def _t2(a0, a1, g1, dis16, bias1, wt, b):
    return pl.pallas_call(
        _t2_body,
        grid=(GRID,),
        in_specs=[
            pl.BlockSpec((BLK, H), lambda i: (i, 0)),
            pl.BlockSpec((BLK, H), lambda i: (i, 0)),
            pl.BlockSpec((BLK, H), lambda i: (i, 0)),
            pl.BlockSpec((BLK, 16), lambda i: (i, 0)),
            pl.BlockSpec((1, H), lambda i: (0, 0)),
            pl.BlockSpec((H, H), lambda i: (0, 0)),
            pl.BlockSpec((1, H), lambda i: (0, 0)),
        ],
        out_specs=pl.BlockSpec((BLK, H), lambda i: (i, 0)),
        out_shape=jax.ShapeDtypeStruct((N, H), jnp.float32),
    )(a0, a1, g1, dis16, bias1, wt, b)


def _t3_body(a0_ref, a1_ref, g2_ref, dis_ref, bias_ref,
             ncw1t_ref, ncb1_ref, ncw2_ref, ncb2_ref,
             w1at_ref, b1e_ref, w1bt_ref,
             nlog_ref, ap_ref, bm_ref):
    dis = dis_ref[...][:, 0:1]
    h = dis * (a0_ref[...] + a1_ref[...] + g2_ref[...]) + bias_ref[...]
    z = jnp.maximum(
        jnp.dot(h, ncw1t_ref[...], preferred_element_type=jnp.float32)
        + ncb1_ref[...], 0.0)
    nlog = jnp.sum(z * ncw2_ref[...], axis=1, keepdims=True) + ncb2_ref[...]
    nlog_ref[...] = jnp.broadcast_to(nlog, (BLK, 16))
    ap_ref[...] = jnp.dot(h, w1at_ref[...],
                          preferred_element_type=jnp.float32) + b1e_ref[...]
    bm_ref[...] = jnp.dot(h, w1bt_ref[...], preferred_element_type=jnp.float32)


def _t3(a0, a1, g2, dis16, bias2, ncw1t, ncb1, ncw2, ncb2, w1at, b1e, w1bt):
    return pl.pallas_call(
        _t3_body,
        grid=(GRID,),
        in_specs=[
            pl.BlockSpec((BLK, H), lambda i: (i, 0)),
            pl.BlockSpec((BLK, H), lambda i: (i, 0)),
            pl.BlockSpec((BLK, H), lambda i: (i, 0)),
            pl.BlockSpec((BLK, 16), lambda i: (i, 0)),
            pl.BlockSpec((1, H), lambda i: (0, 0)),
            pl.BlockSpec((H, H), lambda i: (0, 0)),
            pl.BlockSpec((1, H), lambda i: (0, 0)),
            pl.BlockSpec((1, H), lambda i: (0, 0)),
            pl.BlockSpec((1, 1), lambda i: (0, 0)),
            pl.BlockSpec((H, 2 * H), lambda i: (0, 0)),
            pl.BlockSpec((1, 2 * H), lambda i: (0, 0)),
            pl.BlockSpec((H, 2 * H), lambda i: (0, 0)),
        ],
        out_specs=[
            pl.BlockSpec((BLK, 16), lambda i: (i, 0)),
            pl.BlockSpec((BLK, 2 * H), lambda i: (i, 0)),
            pl.BlockSpec((BLK, 2 * H), lambda i: (i, 0)),
        ],
        out_shape=[
            jax.ShapeDtypeStruct((N, 16), jnp.float32),
            jax.ShapeDtypeStruct((N, 2 * H), jnp.float32),
            jax.ShapeDtypeStruct((N, 2 * H), jnp.float32),
        ],
    )(a0, a1, g2, dis16, bias2, ncw1t, ncb1, ncw2, ncb2, w1at, b1e, w1bt)


CL = N + EH
CLR = 1330
CLP = CLR * 128


def _soft_body(cl_ref, cb_ref, out_ref, msel_ref):
    msel_ref[...] = jnp.full((CLR, 128), 1e30, jnp.float32)

    def _m(b, _):
        cb = cb_ref[...]
        mb = jnp.max(jnp.where(cb == b, cl_ref[...], -1e30))
        msel_ref[...] = jnp.where(cb == b, mb, msel_ref[...])
        return 0
    lax.fori_loop(0, B, _m, 0)

    out_ref[...] = jnp.exp(cl_ref[...] - msel_ref[...])

    def _s(b, _):
        cb = cb_ref[...]
        sb = jnp.sum(jnp.where(cb == b, out_ref[...], 0.0))
        msel_ref[...] = jnp.where(cb == b, 1.0 / sb, msel_ref[...])
        return 0
    lax.fori_loop(0, B, _s, 0)

    out_ref[...] = out_ref[...] * jnp.where(cb_ref[...] < B, msel_ref[...], 0.0)


def _soft(cl2, cb2):
    return pl.pallas_call(
        _soft_body,
        grid=(1,),
        in_specs=[
            pl.BlockSpec((CLR, 128), lambda i: (0, 0)),
            pl.BlockSpec((CLR, 128), lambda i: (0, 0)),
        ],
        out_specs=pl.BlockSpec((CLR, 128), lambda i: (0, 0)),
        out_shape=jax.ShapeDtypeStruct((CLR, 128), jnp.float32),
        scratch_shapes=[pltpu.VMEM((CLR, 128), jnp.float32)],
    )(cl2, cb2)


def kernel(x, edge_index, edge_attr, info_batch,
           W_g1, b_g1, bias_g1, W_g2, b_g2, bias_g2,
           nc_W1, nc_b1, nc_W2, nc_b2, ec_W1, ec_b1, ec_W2, ec_b2):
    row = edge_index[0]
    col = edge_index[1]
    pad = EHP - EH
    n0p = jnp.concatenate([row[::2], jnp.zeros((pad,), jnp.int32)])
    n1p = jnp.concatenate([col[::2], jnp.zeros((pad,), jnp.int32)])

    degout = _sc_deg(col)
    deg0, deg1 = degout[:N], degout[NP:NP + N]

    dis16, g1 = _t1(deg0, deg1, x, W_g1.T, b_g1[None, :])

    acc1 = _sc_prop(g1, row, col)
    g2 = _t2(acc1[:N], acc1[NP:NP + N], g1, dis16, bias_g1[None, :],
             W_g2.T, b_g2[None, :])

    acc2 = _sc_prop(g2, row, col)
    nlog16, ap, bm = _t3(acc2[:N], acc2[NP:NP + N], g2, dis16, bias_g2[None, :],
                         nc_W1.T, nc_b1[None, :], nc_W2, nc_b2[None, :],
                         ec_W1[:, :H].T, ec_b1[None, :], ec_W1[:, H:].T)

    elog, ebat = _sc_edge(ap, bm, n0p, n1p, info_batch, ec_W2[0],
                          jnp.pad(ec_b2, (0, 15)))

    cl = jnp.concatenate([nlog16[:, 0], elog[:EH],
                          jnp.zeros((CLP - CL,), jnp.float32)])
    cb = jnp.concatenate([info_batch, ebat[:EH],
                          jnp.full((CLP - CL,), B, jnp.int32)])
    soft = _soft(cl.reshape(CLR, 128), cb.reshape(CLR, 128)).reshape(-1)
    return (soft[:N, None], soft[N:CL, None])
